# Initial kernel scaffold; baseline (speedup 1.0000x reference)
import jax
import jax.numpy as jnp
from jax.experimental import pallas as pl


def kernel(x, edge_index, W1, b1, W2, b2):
    raise NotImplementedError("write your pallas kernel here")


# speedup vs baseline: 4.3230x; 4.3230x over previous
import functools

import jax
import jax.numpy as jnp
from jax import lax
from jax.experimental import pallas as pl
from jax.experimental.pallas import tpu as pltpu
from jax.experimental.pallas import tpu_sc as plsc

N = 10000
IN_CH = 256
HIDDEN = 256
NUM_CLASSES = 64

NC = 2
NS = 16
B = 128
FG = 128
ACC_N = 10240
TPS = ACC_N // NS
BM = 1000


def _mesh():
    return plsc.VectorSubcoreMesh(
        core_axis_name="c", subcore_axis_name="s", num_cores=NC, num_subcores=NS
    )


def _fill_zero(ref, nrows, width):
    z = jnp.zeros((16,), jnp.float32)
    per_row = width // 16

    def row(i, carry):
        for j in range(per_row):
            ref[i, pl.ds(j * 16, 16)] = z
        return carry

    lax.fori_loop(0, nrows, row, 0)


PR_HALF = ACC_N // NC
PR_R = PR_HALF + 128
PR_T = PR_R // NS


def _prop_body(NB, NPASS, table, src, dst, out, idx_s, idx_d, idx_t,
               buf_a, buf_b, sem_a, sem_b, acc):
    c = lax.axis_index("c")
    s = lax.axis_index("s")
    t0 = s * PR_T
    rows = pl.ds(t0, PR_T)
    pltpu.sync_copy(src.at[c, s], idx_s)
    pltpu.sync_copy(dst.at[c, s], idx_d)

    for k in range(NPASS):
        r = k if NPASS == NC else c
        base = r * PR_HALF

        def xform(b, carry):
            for g in range(B // 16):
                sl = pl.ds(g * 16, 16)
                idx = idx_d[b, sl]
                local = idx - base
                inb = (local >= 0) & (local < PR_HALF)
                idx_t[b, sl] = jnp.where(inb, local,
                                         PR_HALF + (idx & 127))
            return carry

        lax.fori_loop(0, NB, xform, 0)

        _fill_zero(buf_a, B, FG)
        pltpu.sync_copy(buf_a, acc.at[pl.ds(t0, B)])
        pltpu.sync_copy(buf_a, acc.at[pl.ds(t0 + B, B)])
        pltpu.sync_copy(buf_a.at[pl.ds(0, PR_T - 2 * B)],
                        acc.at[pl.ds(t0 + 2 * B, PR_T - 2 * B)])
        plsc.subcore_barrier()

        pltpu.make_async_copy(table.at[idx_s.at[0]], buf_a, sem_a).start()

        def step(t, carry):
            b0 = 2 * t
            b1 = b0 + 1
            pltpu.make_async_copy(table.at[idx_s.at[b1]], buf_b,
                                  sem_b).start()
            pltpu.make_async_copy(table.at[idx_s.at[b0]], buf_a,
                                  sem_a).wait()
            pltpu.sync_copy(buf_a, acc.at[idx_t.at[b0]], add=True)

            @pl.when(b0 + 2 < NB)
            def _():
                pltpu.make_async_copy(table.at[idx_s.at[b0 + 2]], buf_a,
                                      sem_a).start()

            pltpu.make_async_copy(table.at[idx_s.at[b1]], buf_b,
                                  sem_b).wait()
            pltpu.sync_copy(buf_b, acc.at[idx_t.at[b1]], add=True)
            return carry

        lax.fori_loop(0, NB // 2, step, 0)
        plsc.subcore_barrier()
        pltpu.sync_copy(acc.at[rows], out.at[c, k, rows])
        plsc.subcore_barrier()


def _make_prop(NB, NPASS):
    return pl.kernel(
        functools.partial(_prop_body, NB, NPASS),
        out_type=jax.ShapeDtypeStruct((NC, NPASS, PR_R, FG), jnp.float32),
        scratch_types=[
            pltpu.VMEM((NB, B), jnp.int32),
            pltpu.VMEM((NB, B), jnp.int32),
            pltpu.VMEM((NB, B), jnp.int32),
            pltpu.VMEM((B, FG), jnp.float32),
            pltpu.VMEM((B, FG), jnp.float32),
            pltpu.SemaphoreType.DMA,
            pltpu.SemaphoreType.DMA,
            pltpu.VMEM_SHARED((PR_R, FG), jnp.float32),
        ],
        mesh=_mesh(),
    )


DEG_HALF = ACC_N // NC
DEG_R = DEG_HALF + 128
DEG_T = DEG_R // NS


def _deg_body(NB, dst, out, idx_d, idx2, ones_v, zbuf, acc):
    c = lax.axis_index("c")
    s = lax.axis_index("s")
    base = c * DEG_HALF
    pltpu.sync_copy(dst.at[s], idx_d)

    ones = jnp.ones((16,), jnp.float32)
    z = jnp.zeros((16,), jnp.float32)

    def fill(i, carry):
        ones_v[i, pl.ds(0, 16)] = ones
        zbuf[i, pl.ds(0, 16)] = z
        return carry

    lax.fori_loop(0, B, fill, 0)

    t0 = s * DEG_T
    pltpu.sync_copy(zbuf, acc.at[pl.ds(t0, B)])
    pltpu.sync_copy(zbuf, acc.at[pl.ds(t0 + B, B)])
    pltpu.sync_copy(zbuf.at[pl.ds(0, DEG_T - 2 * B)],
                    acc.at[pl.ds(t0 + 2 * B, DEG_T - 2 * B)])
    plsc.subcore_barrier()

    def step(b, carry):
        for k in range(B // 16):
            idx = idx_d[b, pl.ds(k * 16, 16)]
            local = idx - base
            inb = (local >= 0) & (local < DEG_HALF)
            idx2[0, pl.ds(k * 16, 16)] = jnp.where(inb, local, DEG_HALF)
        pltpu.sync_copy(ones_v, acc.at[idx2.at[0]], add=True)
        return carry

    lax.fori_loop(0, NB, step, 0)
    plsc.subcore_barrier()
    rows = pl.ds(t0, DEG_T)
    pltpu.sync_copy(acc.at[rows], out.at[c, rows])


def _make_deg(NB):
    return pl.kernel(
        functools.partial(_deg_body, NB),
        out_type=jax.ShapeDtypeStruct((NC, DEG_R, 16), jnp.float32),
        scratch_types=[
            pltpu.VMEM((NB, B), jnp.int32),
            pltpu.VMEM((1, B), jnp.int32),
            pltpu.VMEM((B, 16), jnp.float32),
            pltpu.VMEM((B, 16), jnp.float32),
            pltpu.VMEM_SHARED((DEG_R, 16), jnp.float32),
        ],
        mesh=_mesh(),
    )


RB = 1280


def _dinv_body(deg_ref, out_ref):
    out_ref[...] = lax.rsqrt(deg_ref[...] + 1.0)


def _dinv_kernel(deg_row):
    return pl.pallas_call(
        _dinv_body,
        grid=(ACC_N // RB,),
        in_specs=[pl.BlockSpec((1, RB), lambda i: (0, i))],
        out_specs=pl.BlockSpec((1, RB), lambda i: (0, i)),
        out_shape=jax.ShapeDtypeStruct((1, ACC_N), jnp.float32),
    )(deg_row)


def _stage_b_body(x_ref, w_ref, dinv_ref, xws_ref):
    dinv = dinv_ref[...]
    xw = jnp.dot(x_ref[...], w_ref[0], preferred_element_type=jnp.float32)
    xws_ref[0] = dinv * xw


def _stage_b(x, W1p, dinv):
    return pl.pallas_call(
        _stage_b_body,
        grid=(N // BM, NC),
        in_specs=[
            pl.BlockSpec((BM, IN_CH), lambda i, j: (i, 0)),
            pl.BlockSpec((1, IN_CH, FG), lambda i, j: (j, 0, 0)),
            pl.BlockSpec((BM, 1), lambda i, j: (i, 0)),
        ],
        out_specs=pl.BlockSpec((1, BM, FG), lambda i, j: (j, i, 0)),
        out_shape=jax.ShapeDtypeStruct((NC, N, FG), jnp.float32),
    )(x, W1p, dinv)


def _stage_d_body(s1_ref, xws_ref, dinv_ref, b1_ref, w2_ref, out_ref):
    dv = dinv_ref[...]
    h0 = jnp.maximum(dv * (s1_ref[0] + xws_ref[0]) + b1_ref[0], 0.0)
    h1 = jnp.maximum(dv * (s1_ref[1] + xws_ref[1]) + b1_ref[1], 0.0)
    hw2 = (jnp.dot(h0, w2_ref[0:FG], preferred_element_type=jnp.float32)
           + jnp.dot(h1, w2_ref[FG:HIDDEN],
                     preferred_element_type=jnp.float32))
    hws2 = dv * hw2
    out_ref[...] = jnp.concatenate(
        [hws2, jnp.zeros((BM, FG - NUM_CLASSES), jnp.float32)], axis=1)


def _stage_d(s1, xws, dinv, b1r, W2):
    return pl.pallas_call(
        _stage_d_body,
        grid=(N // BM,),
        in_specs=[
            pl.BlockSpec((NC, BM, FG), lambda i: (0, i, 0)),
            pl.BlockSpec((NC, BM, FG), lambda i: (0, i, 0)),
            pl.BlockSpec((BM, 1), lambda i: (i, 0)),
            pl.BlockSpec((NC, 1, FG), lambda i: (0, 0, 0)),
            pl.BlockSpec((HIDDEN, NUM_CLASSES), lambda i: (0, 0)),
        ],
        out_specs=pl.BlockSpec((BM, FG), lambda i: (i, 0)),
        out_shape=jax.ShapeDtypeStruct((N, FG), jnp.float32),
    )(s1, xws, dinv, b1r, W2)


def _stage_f_body(p_ref, hws_ref, dinv_ref, b2_ref, out_ref):
    dv = dinv_ref[...]
    hws = hws_ref[:, 0:NUM_CLASSES]
    out_ref[...] = dv * (p_ref[:, 0:NUM_CLASSES] + hws) + b2_ref[...]


def _stage_f(p, hws2p, dinv, b2r):
    return pl.pallas_call(
        _stage_f_body,
        grid=(N // BM,),
        in_specs=[
            pl.BlockSpec((BM, FG), lambda i: (i, 0)),
            pl.BlockSpec((BM, FG), lambda i: (i, 0)),
            pl.BlockSpec((BM, 1), lambda i: (i, 0)),
            pl.BlockSpec((1, NUM_CLASSES), lambda i: (0, 0)),
        ],
        out_specs=pl.BlockSpec((BM, NUM_CLASSES), lambda i: (i, 0)),
        out_shape=jax.ShapeDtypeStruct((N, NUM_CLASSES), jnp.float32),
    )(p, hws2p, dinv, b2r)


def kernel(x, edge_index, W1, b1, W2, b2):
    E = edge_index.shape[1]
    chunk = NC * NS * B
    PE = -(-E // chunk) * chunk
    NB2 = PE // chunk
    NB1 = PE // (NS * B)

    ei = edge_index.astype(jnp.int32)
    src = jnp.concatenate([ei[0], jnp.zeros((PE - E,), jnp.int32)])
    dst = jnp.concatenate([ei[1], jnp.full((PE - E,), N, jnp.int32)])

    offs = (jnp.arange(NC, dtype=jnp.int32) * N).reshape(NC, 1, 1, 1)
    src1 = src.reshape(1, NS, NB1, B) + offs
    src2 = jnp.broadcast_to(src.reshape(1, NS, NB1, B), (NC, NS, NB1, B))
    dst1 = jnp.broadcast_to(dst.reshape(1, NS, NB1, B), (NC, NS, NB1, B))

    dst1m = dst.reshape(NS, NB1, B)
    degp = _make_deg(NB1)(dst1m)
    deg_row = jnp.concatenate(
        [degp[0, :DEG_HALF, 0], degp[1, :DEG_HALF, 0]])[None, :]
    dinv_row = _dinv_kernel(deg_row)
    dinv = dinv_row[0, :N][:, None]

    W1p = W1.reshape(IN_CH, NC, FG).transpose(1, 0, 2)
    xws = _stage_b(x, W1p, dinv)

    table1 = xws.reshape(NC * N, FG)
    s1 = _make_prop(NB1, NC)(table1, src1, dst1)
    s1v = s1[:, :, :PR_HALF, :].reshape(NC, ACC_N, FG)

    hws2p = _stage_d(s1v, xws, dinv, b1.reshape(NC, 1, FG), W2)

    s2 = _make_prop(NB1, 1)(hws2p, src2, dst1)
    s2v = s2[:, 0, :PR_HALF, :].reshape(ACC_N, FG)

    return _stage_f(s2v, hws2p, dinv, b2.reshape(1, NUM_CLASSES))
